# baseline (speedup 1.0000x reference)
import functools

import jax
import jax.numpy as jnp
import numpy as np
from jax import lax
from jax.experimental import pallas as pl
from jax.experimental.pallas import tpu as pltpu
from jax.experimental.pallas import tpu_sc as plsc

E = 8
K = 2
_EPS = float(np.finfo(np.float64).eps)
_TM = 1024


def _gating_kernel(x_ref, wg_ref, i12_ref, g12_ref, loss_ref):
    x = x_ref[...]
    wg = wg_ref[...]
    logits = jnp.dot(x, wg, preferred_element_type=jnp.float32)
    cols = jax.lax.broadcasted_iota(jnp.int32, logits.shape, 1)
    m1 = jnp.max(logits, axis=1, keepdims=True)
    i1 = jnp.min(jnp.where(logits == m1, cols, E), axis=1, keepdims=True)
    sel1 = cols == i1
    masked = jnp.where(sel1, -jnp.inf, logits)
    m2 = jnp.max(masked, axis=1, keepdims=True)
    i2 = jnp.min(jnp.where(masked == m2, cols, E), axis=1, keepdims=True)
    sel2 = cols == i2
    e2 = jnp.exp(m2 - m1)
    g1 = 1.0 / (1.0 + e2)
    g2 = e2 / (1.0 + e2)
    gates = jnp.where(sel1, g1, 0.0) + jnp.where(sel2, g2, 0.0)
    i12_ref[...] = jnp.concatenate([i1, i2], axis=1)
    g12_ref[...] = jnp.concatenate([g1, g2], axis=1)
    imp = jnp.sum(gates, axis=0)
    load = jnp.sum((gates > 0.0).astype(jnp.float32), axis=0)

    def cv2(v):
        mean = jnp.mean(v)
        var = jnp.sum((v - mean) ** 2) / (E - 1)
        return var / (mean * mean + 1e-10)

    loss_ref[...] = jnp.reshape((cv2(imp) + cv2(load)) * 0.01, (1, 1))


def _grouped_kernel(items_ref, xs_ref, w1_ref, b1_ref, w2_ref, b2_ref, ys_ref):
    i = pl.program_id(0)
    lo = items_ref[i, 2]
    hi = items_ref[i, 3]

    @pl.when(lo < hi)
    def _():
        x = xs_ref[...]
        h = jnp.dot(x, w1_ref[0], preferred_element_type=jnp.float32)
        h = jnp.maximum(h + b1_ref[0, 0], 0.0)
        o = jnp.dot(h, w2_ref[0], preferred_element_type=jnp.float32)
        o = o + b2_ref[0, 0]
        o = jax.nn.softmax(o, axis=-1)
        contrib = jnp.exp(o)
        rows = jax.lax.broadcasted_iota(jnp.int32, (contrib.shape[0], 1), 0)
        contrib = jnp.where((rows >= lo) & (rows < hi), contrib, 0.0)

        @pl.when(lo == 0)
        def _():
            ys_ref[...] = contrib

        @pl.when(lo > 0)
        def _():
            ys_ref[...] += contrib


def _final_kernel(c_ref, g_ref, y_ref):
    s = (g_ref[0, 0][:, None] * c_ref[0]
         + g_ref[1, 0][:, None] * c_ref[1])
    s = jnp.where(s == 0.0, _EPS, s)
    y_ref[...] = jnp.log(s)


def _sc_gather(table, idx):
    R = idx.shape[0]
    D = table.shape[1]
    info = plsc.get_sparse_core_info()
    NC, NS = info.num_cores, info.num_subcores
    NW = NC * NS
    rows_per_w = R // NW
    CH = min(64, rows_per_w)
    n_ch = rows_per_w // CH
    mesh = plsc.VectorSubcoreMesh(core_axis_name="c", subcore_axis_name="s")

    @functools.partial(
        pl.kernel, mesh=mesh,
        out_type=jax.ShapeDtypeStruct((R, D), jnp.float32),
        scratch_types=[
            pltpu.VMEM((CH,), jnp.int32),
            pltpu.VMEM((CH, D), jnp.float32),
            pltpu.SemaphoreType.DMA,
        ],
    )
    def k(table_hbm, idx_hbm, out_hbm, idx_v, rows_v, sem):
        wid = lax.axis_index("s") * NC + lax.axis_index("c")
        base = wid * rows_per_w
        for c in range(n_ch):
            off = base + c * CH
            pltpu.sync_copy(idx_hbm.at[pl.ds(off, CH)], idx_v)
            pltpu.async_copy(table_hbm.at[idx_v], rows_v, sem).wait()
            pltpu.sync_copy(rows_v, out_hbm.at[pl.ds(off, CH)])

    return k(table, idx)


def _sc_dispatch(x, srcidx, pos):
    P = pos.shape[0]
    D = x.shape[1]
    info = plsc.get_sparse_core_info()
    NC, NS = info.num_cores, info.num_subcores
    NW = NC * NS
    rows_per_w = P // NW
    CH = min(64, rows_per_w)
    n_ch = rows_per_w // CH
    mesh = plsc.VectorSubcoreMesh(core_axis_name="c", subcore_axis_name="s")

    @functools.partial(
        pl.kernel, mesh=mesh,
        out_type=jax.ShapeDtypeStruct((P, D), jnp.float32),
        scratch_types=[
            pltpu.VMEM((CH,), jnp.int32),
            pltpu.VMEM((CH,), jnp.int32),
            pltpu.VMEM((CH, D), jnp.float32),
            pltpu.SemaphoreType.DMA,
        ],
    )
    def k(x_hbm, src_hbm, pos_hbm, out_hbm, sidx_v, didx_v, rows_v, sem):
        wid = lax.axis_index("s") * NC + lax.axis_index("c")
        base = wid * rows_per_w
        for c in range(n_ch):
            off = base + c * CH
            pltpu.sync_copy(src_hbm.at[pl.ds(off, CH)], sidx_v)
            pltpu.sync_copy(pos_hbm.at[pl.ds(off, CH)], didx_v)
            pltpu.async_copy(x_hbm.at[sidx_v], rows_v, sem).wait()
            pltpu.sync_copy(rows_v, out_hbm.at[didx_v])

    return k(x, srcidx, pos)


def kernel(x, w_gate, W1, b1, W2, b2):
    B, D = x.shape
    H = W1.shape[2]
    TM = _TM
    P = K * B
    T = P // TM
    NI = T + E - 1

    i12, g12, loss2 = pl.pallas_call(
        _gating_kernel,
        out_shape=[
            jax.ShapeDtypeStruct((B, K), jnp.int32),
            jax.ShapeDtypeStruct((B, K), jnp.float32),
            jax.ShapeDtypeStruct((1, 1), jnp.float32),
        ],
    )(x, w_gate)

    flat_e = i12.reshape(-1)
    oh = jax.nn.one_hot(flat_e, E, dtype=jnp.float32)
    csum = jnp.cumsum(oh, axis=0)
    counts = csum[-1].astype(jnp.int32)
    offsets = jnp.concatenate(
        [jnp.zeros((1,), jnp.int32), jnp.cumsum(counts)[:-1].astype(jnp.int32)])
    pos = jnp.sum(
        oh * (csum - 1.0 + offsets.astype(jnp.float32)[None, :]),
        axis=1).astype(jnp.int32)
    allpos = jnp.concatenate([pos[0::2], pos[1::2]])
    src_tok = (jnp.arange(P, dtype=jnp.int32) // K)
    tile_lo = (jnp.arange(T, dtype=jnp.int32) * TM)[None, :]
    glo = jnp.maximum(offsets[:, None], tile_lo)
    ghi = jnp.minimum((offsets + counts)[:, None], tile_lo + TM)
    flag = (glo < ghi).reshape(-1)
    order = jnp.cumsum(flag.astype(jnp.int32)) - 1
    sci = jnp.where(flag, order, NI)
    e_flat = jnp.repeat(jnp.arange(E, dtype=jnp.int32), T)
    t_flat = jnp.tile(jnp.arange(T, dtype=jnp.int32), E)
    vals = jnp.stack(
        [t_flat, e_flat,
         glo.reshape(-1) - t_flat * TM,
         ghi.reshape(-1) - t_flat * TM], axis=1)
    defaults = jnp.array([T - 1, E - 1, TM, TM], jnp.int32)
    items = jnp.broadcast_to(defaults, (NI, 4)).at[sci].set(vals, mode='drop')

    xs = _sc_dispatch(x, src_tok, pos)

    grid_spec = pltpu.PrefetchScalarGridSpec(
        num_scalar_prefetch=1,
        grid=(NI,),
        in_specs=[
            pl.BlockSpec((TM, D), lambda i, it: (it[i, 0], 0)),
            pl.BlockSpec((1, D, H), lambda i, it: (it[i, 1], 0, 0)),
            pl.BlockSpec((1, 1, H), lambda i, it: (it[i, 1], 0, 0)),
            pl.BlockSpec((1, H, D), lambda i, it: (it[i, 1], 0, 0)),
            pl.BlockSpec((1, 1, D), lambda i, it: (it[i, 1], 0, 0)),
        ],
        out_specs=pl.BlockSpec((TM, D), lambda i, it: (it[i, 0], 0)),
    )
    ys = pl.pallas_call(
        _grouped_kernel,
        grid_spec=grid_spec,
        out_shape=jax.ShapeDtypeStruct((P, D), jnp.float32),
    )(items, xs, W1, b1.reshape(E, 1, H), W2, b2.reshape(E, 1, D))

    comb2 = _sc_gather(ys, allpos)

    TMF = 256
    y = pl.pallas_call(
        _final_kernel,
        grid=(B // TMF,),
        in_specs=[
            pl.BlockSpec((K, TMF, D), lambda t: (0, t, 0)),
            pl.BlockSpec((K, 1, TMF), lambda t: (0, 0, t)),
        ],
        out_specs=pl.BlockSpec((TMF, D), lambda t: (t, 0)),
        out_shape=jax.ShapeDtypeStruct((B, D), jnp.float32),
    )(comb2.reshape(K, B, D), g12.T.reshape(K, 1, B))

    return (y, loss2[0, 0])


# speedup vs baseline: 1.1736x; 1.1736x over previous
import functools

import jax
import jax.numpy as jnp
import numpy as np
from jax import lax
from jax.experimental import pallas as pl
from jax.experimental.pallas import tpu as pltpu
from jax.experimental.pallas import tpu_sc as plsc

E = 8
K = 2
_EPS = float(np.finfo(np.float64).eps)
_TM = 512


def _gating_kernel(x_ref, wg_ref, i12_ref, g12_ref, loss_ref):
    x = x_ref[...]
    wg = wg_ref[...]
    logits = jnp.dot(x, wg, preferred_element_type=jnp.float32)
    cols = jax.lax.broadcasted_iota(jnp.int32, logits.shape, 1)
    m1 = jnp.max(logits, axis=1, keepdims=True)
    i1 = jnp.min(jnp.where(logits == m1, cols, E), axis=1, keepdims=True)
    sel1 = cols == i1
    masked = jnp.where(sel1, -jnp.inf, logits)
    m2 = jnp.max(masked, axis=1, keepdims=True)
    i2 = jnp.min(jnp.where(masked == m2, cols, E), axis=1, keepdims=True)
    sel2 = cols == i2
    e2 = jnp.exp(m2 - m1)
    g1 = 1.0 / (1.0 + e2)
    g2 = e2 / (1.0 + e2)
    gates = jnp.where(sel1, g1, 0.0) + jnp.where(sel2, g2, 0.0)
    i12_ref[...] = jnp.concatenate([i1, i2], axis=1)
    g12_ref[...] = jnp.concatenate([g1, g2], axis=1)
    imp = jnp.sum(gates, axis=0)
    load = jnp.sum((gates > 0.0).astype(jnp.float32), axis=0)

    def cv2(v):
        mean = jnp.mean(v)
        var = jnp.sum((v - mean) ** 2) / (E - 1)
        return var / (mean * mean + 1e-10)

    loss_ref[...] = jnp.reshape((cv2(imp) + cv2(load)) * 0.01, (1, 1))


def _grouped_kernel(items_ref, xs_ref, w1_ref, b1_ref, w2_ref, b2_ref, ys_ref):
    i = pl.program_id(0)
    lo = items_ref[i, 2]
    hi = items_ref[i, 3]

    @pl.when(lo < hi)
    def _():
        x = xs_ref[...]
        h = jnp.dot(x, w1_ref[0], preferred_element_type=jnp.float32)
        h = jnp.maximum(h + b1_ref[0, 0], 0.0)
        o = jnp.dot(h, w2_ref[0], preferred_element_type=jnp.float32)
        o = o + b2_ref[0, 0]
        o = jax.nn.softmax(o, axis=-1)
        contrib = jnp.exp(o)
        rows = jax.lax.broadcasted_iota(jnp.int32, (contrib.shape[0], 1), 0)
        contrib = jnp.where((rows >= lo) & (rows < hi), contrib, 0.0)

        @pl.when(lo == 0)
        def _():
            ys_ref[...] = contrib

        @pl.when(lo > 0)
        def _():
            ys_ref[...] += contrib


def _final_kernel(c_ref, g_ref, y_ref):
    s = (g_ref[0, 0][:, None] * c_ref[0]
         + g_ref[1, 0][:, None] * c_ref[1])
    s = jnp.where(s == 0.0, _EPS, s)
    y_ref[...] = jnp.log(s)


def _sc_gather(table, idx):
    R = idx.shape[0]
    D = table.shape[1]
    info = plsc.get_sparse_core_info()
    NC, NS = info.num_cores, info.num_subcores
    NW = NC * NS
    rows_per_w = R // NW
    CH = min(64, rows_per_w)
    n_ch = rows_per_w // CH
    mesh = plsc.VectorSubcoreMesh(core_axis_name="c", subcore_axis_name="s")

    @functools.partial(
        pl.kernel, mesh=mesh,
        out_type=jax.ShapeDtypeStruct((R, D), jnp.float32),
        scratch_types=[
            pltpu.VMEM((CH,), jnp.int32),
            pltpu.VMEM((CH, D), jnp.float32),
            pltpu.SemaphoreType.DMA,
        ],
    )
    def k(table_hbm, idx_hbm, out_hbm, idx_v, rows_v, sem):
        wid = lax.axis_index("s") * NC + lax.axis_index("c")
        base = wid * rows_per_w
        for c in range(n_ch):
            off = base + c * CH
            pltpu.sync_copy(idx_hbm.at[pl.ds(off, CH)], idx_v)
            pltpu.async_copy(table_hbm.at[idx_v], rows_v, sem).wait()
            pltpu.sync_copy(rows_v, out_hbm.at[pl.ds(off, CH)])

    return k(table, idx)


def _sc_dispatch(x, srcidx, pos):
    P = pos.shape[0]
    D = x.shape[1]
    info = plsc.get_sparse_core_info()
    NC, NS = info.num_cores, info.num_subcores
    NW = NC * NS
    rows_per_w = P // NW
    CH = min(64, rows_per_w)
    n_ch = rows_per_w // CH
    mesh = plsc.VectorSubcoreMesh(core_axis_name="c", subcore_axis_name="s")

    @functools.partial(
        pl.kernel, mesh=mesh,
        out_type=jax.ShapeDtypeStruct((P, D), jnp.float32),
        scratch_types=[
            pltpu.VMEM((CH,), jnp.int32),
            pltpu.VMEM((CH,), jnp.int32),
            pltpu.VMEM((CH, D), jnp.float32),
            pltpu.SemaphoreType.DMA,
        ],
    )
    def k(x_hbm, src_hbm, pos_hbm, out_hbm, sidx_v, didx_v, rows_v, sem):
        wid = lax.axis_index("s") * NC + lax.axis_index("c")
        base = wid * rows_per_w
        for c in range(n_ch):
            off = base + c * CH
            pltpu.sync_copy(src_hbm.at[pl.ds(off, CH)], sidx_v)
            pltpu.sync_copy(pos_hbm.at[pl.ds(off, CH)], didx_v)
            pltpu.async_copy(x_hbm.at[sidx_v], rows_v, sem).wait()
            pltpu.sync_copy(rows_v, out_hbm.at[didx_v])

    return k(x, srcidx, pos)


def kernel(x, w_gate, W1, b1, W2, b2):
    B, D = x.shape
    H = W1.shape[2]
    TM = _TM
    P = K * B
    T = P // TM
    NI = T + E - 1

    i12, g12, loss2 = pl.pallas_call(
        _gating_kernel,
        out_shape=[
            jax.ShapeDtypeStruct((B, K), jnp.int32),
            jax.ShapeDtypeStruct((B, K), jnp.float32),
            jax.ShapeDtypeStruct((1, 1), jnp.float32),
        ],
    )(x, w_gate)

    flat_e = i12.reshape(-1)
    oh = jax.nn.one_hot(flat_e, E, dtype=jnp.float32)
    csum = jnp.cumsum(oh, axis=0)
    counts = csum[-1].astype(jnp.int32)
    offsets = jnp.concatenate(
        [jnp.zeros((1,), jnp.int32), jnp.cumsum(counts)[:-1].astype(jnp.int32)])
    pos = jnp.sum(
        oh * (csum - 1.0 + offsets.astype(jnp.float32)[None, :]),
        axis=1).astype(jnp.int32)
    allpos = jnp.concatenate([pos[0::2], pos[1::2]])
    src_tok = (jnp.arange(P, dtype=jnp.int32) // K)
    tile_lo = (jnp.arange(T, dtype=jnp.int32) * TM)[None, :]
    glo = jnp.maximum(offsets[:, None], tile_lo)
    ghi = jnp.minimum((offsets + counts)[:, None], tile_lo + TM)
    flag = (glo < ghi).reshape(-1)
    order = jnp.cumsum(flag.astype(jnp.int32)) - 1
    sci = jnp.where(flag, order, NI)
    e_flat = jnp.repeat(jnp.arange(E, dtype=jnp.int32), T)
    t_flat = jnp.tile(jnp.arange(T, dtype=jnp.int32), E)
    vals = jnp.stack(
        [t_flat, e_flat,
         glo.reshape(-1) - t_flat * TM,
         ghi.reshape(-1) - t_flat * TM], axis=1)
    defaults = jnp.array([T - 1, E - 1, TM, TM], jnp.int32)
    items = jnp.broadcast_to(defaults, (NI, 4)).at[sci].set(vals, mode='drop')

    xs = _sc_dispatch(x, src_tok, pos)

    grid_spec = pltpu.PrefetchScalarGridSpec(
        num_scalar_prefetch=1,
        grid=(NI,),
        in_specs=[
            pl.BlockSpec((TM, D), lambda i, it: (it[i, 0], 0)),
            pl.BlockSpec((1, D, H), lambda i, it: (it[i, 1], 0, 0)),
            pl.BlockSpec((1, 1, H), lambda i, it: (it[i, 1], 0, 0)),
            pl.BlockSpec((1, H, D), lambda i, it: (it[i, 1], 0, 0)),
            pl.BlockSpec((1, 1, D), lambda i, it: (it[i, 1], 0, 0)),
        ],
        out_specs=pl.BlockSpec((TM, D), lambda i, it: (it[i, 0], 0)),
    )
    ys = pl.pallas_call(
        _grouped_kernel,
        grid_spec=grid_spec,
        out_shape=jax.ShapeDtypeStruct((P, D), jnp.float32),
    )(items, xs, W1, b1.reshape(E, 1, H), W2, b2.reshape(E, 1, D))

    comb2 = _sc_gather(ys, allpos)

    TMF = 256
    y = pl.pallas_call(
        _final_kernel,
        grid=(B // TMF,),
        in_specs=[
            pl.BlockSpec((K, TMF, D), lambda t: (0, t, 0)),
            pl.BlockSpec((K, 1, TMF), lambda t: (0, 0, t)),
        ],
        out_specs=pl.BlockSpec((TMF, D), lambda t: (t, 0)),
        out_shape=jax.ShapeDtypeStruct((B, D), jnp.float32),
    )(comb2.reshape(K, B, D), g12.T.reshape(K, 1, B))

    return (y, loss2[0, 0])
